# Initial kernel scaffold; baseline (speedup 1.0000x reference)
import jax
import jax.numpy as jnp
from jax.experimental import pallas as pl


def kernel(pts, edges):
    raise NotImplementedError("write your pallas kernel here")


# speedup vs baseline: 26.1114x; 26.1114x over previous
import jax
import jax.numpy as jnp
from jax import lax
from jax.experimental import pallas as pl
from jax.experimental.pallas import tpu as pltpu
from jax.experimental.pallas import tpu_sc as plsc

_K = 10
_N = 8192
_E = 50000


_R = 128


def _dtm_body(pts_blk_ref, ptsT_ref, out_ref):
    a = pts_blk_ref[...]
    bt = ptsT_ref[...]
    sqr = jnp.sum(a * a, axis=1, keepdims=True)
    sqc = jnp.sum(bt * bt, axis=0, keepdims=True)
    ab = a.astype(jnp.bfloat16).astype(jnp.float32)
    btb = bt.astype(jnp.bfloat16).astype(jnp.float32)
    G = (ab[:, 0:1] * btb[0:1, :]
         + ab[:, 1:2] * btb[1:2, :]
         + ab[:, 2:3] * btb[2:3, :])
    D = sqr + sqc - 2.0 * G
    col = lax.broadcasted_iota(jnp.int32, D.shape, 1)
    acc = jnp.zeros((D.shape[0], 1), jnp.float32)
    for t in range(_K + 1):
        m = jnp.min(D, axis=1, keepdims=True)
        if t > 0:
            acc = acc + jnp.maximum(m, 1e-12)
        if t < _K:
            hit = D == m
            first = jnp.min(
                jnp.where(hit, col, jnp.int32(2147483647)), axis=1, keepdims=True
            )
            D = jnp.where(col == first, jnp.float32(jnp.inf), D)
    out_ref[...] = jnp.sqrt(acc * (1.0 / _K))


def _dtm(pts):
    ptsp = jnp.zeros((_N, 8), jnp.float32).at[:, :3].set(pts)
    ptsT = ptsp.T
    out = pl.pallas_call(
        _dtm_body,
        grid=(_N // _R,),
        in_specs=[
            pl.BlockSpec((_R, 8), lambda i: (i, 0)),
            pl.BlockSpec((8, _N), lambda i: (0, 0)),
        ],
        out_specs=pl.BlockSpec((_R, 1), lambda i: (i, 0)),
        out_shape=jax.ShapeDtypeStruct((_N, 1), jnp.float32),
    )(ptsp, ptsT)
    return out[:, 0]


_NW = 32
_EPW = 1568
_EPAD = _NW * _EPW


def _edge_body(tbl_hbm, eu_hbm, ev_hbm, out_hbm, tbl_v, iu_v, iv_v, res_v):
    c = lax.axis_index("c")
    s = lax.axis_index("s")
    wid = s * 2 + c
    base = wid * _EPW
    pltpu.sync_copy(tbl_hbm, tbl_v)
    pltpu.sync_copy(eu_hbm.at[pl.ds(base, _EPW)], iu_v)
    pltpu.sync_copy(ev_hbm.at[pl.ds(base, _EPW)], iv_v)
    for i in range(_EPW // 16):
        u = iu_v[pl.ds(i * 16, 16)]
        v = iv_v[pl.ds(i * 16, 16)]
        xu = plsc.load_gather(tbl_v, [u])
        xv = plsc.load_gather(tbl_v, [v])
        yu = plsc.load_gather(tbl_v, [u + _N])
        yv = plsc.load_gather(tbl_v, [v + _N])
        zu = plsc.load_gather(tbl_v, [u + 2 * _N])
        zv = plsc.load_gather(tbl_v, [v + 2 * _N])
        fu = plsc.load_gather(tbl_v, [u + 3 * _N])
        fv = plsc.load_gather(tbl_v, [v + 3 * _N])
        dx = xu - xv
        dy = yu - yv
        dz = zu - zv
        s2 = dx * dx + dy * dy + dz * dz + 1e-12
        ib = plsc.bitcast(s2, jnp.int32)
        yb = lax.shift_right_logical(ib, jnp.int32(1)) + jnp.int32(0x1FBD1DF5)
        y = plsc.bitcast(yb, jnp.float32)
        y = 0.5 * (y + s2 / y)
        y = 0.5 * (y + s2 / y)
        y = 0.5 * (y + s2 / y)
        res_v[pl.ds(i * 16, 16)] = y + jnp.maximum(fu, fv)
    pltpu.sync_copy(res_v, out_hbm.at[pl.ds(base, _EPW)])


def _edge_call(tbl, eu, ev):
    mesh = plsc.VectorSubcoreMesh(core_axis_name="c", subcore_axis_name="s")
    run = pl.kernel(
        _edge_body,
        out_type=jax.ShapeDtypeStruct((_EPAD,), jnp.float32),
        mesh=mesh,
        compiler_params=pltpu.CompilerParams(needs_layout_passes=False),
        scratch_types=[
            pltpu.VMEM((4 * _N,), jnp.float32),
            pltpu.VMEM((_EPW,), jnp.int32),
            pltpu.VMEM((_EPW,), jnp.int32),
            pltpu.VMEM((_EPW,), jnp.float32),
        ],
    )
    return run(tbl, eu, ev)


def kernel(pts, edges):
    dtm = _dtm(pts)
    tbl = jnp.concatenate([pts, dtm[:, None]], axis=1).T.reshape(-1)
    eu = jnp.zeros((_EPAD,), jnp.int32).at[:_E].set(edges[:, 0])
    ev = jnp.zeros((_EPAD,), jnp.int32).at[:_E].set(edges[:, 1])
    out = _edge_call(tbl, eu, ev)
    return out[:_E]


# speedup vs baseline: 54.2353x; 2.0771x over previous
import jax
import jax.numpy as jnp
from jax import lax
from jax.experimental import pallas as pl
from jax.experimental.pallas import tpu as pltpu
from jax.experimental.pallas import tpu_sc as plsc

_K = 10
_N = 8192
_E = 50000


_R = 128


def _dtm_body(pts_blk_ref, ptsT_ref, out_ref):
    a = pts_blk_ref[...]
    bt = ptsT_ref[...]
    sqr = jnp.sum(a * a, axis=1, keepdims=True)
    sqc = jnp.sum(bt * bt, axis=0, keepdims=True)
    G = jnp.dot(
        a.astype(jnp.bfloat16),
        bt.astype(jnp.bfloat16),
        preferred_element_type=jnp.float32,
    )
    R = a.shape[0]
    INF = jnp.float32(jnp.inf)
    s = [jnp.full((R, 128), INF, jnp.float32) for _ in range(_K + 1)]
    for v in range(_N // 128):
        sl = slice(v * 128, (v + 1) * 128)
        Dv = (sqr + sqc[:, sl]) - 2.0 * G[:, sl]
        mx = [jnp.maximum(s[i], Dv) for i in range(_K)]
        s[0] = jnp.minimum(s[0], Dv)
        for i in range(1, _K + 1):
            s[i] = jnp.minimum(s[i], mx[i - 1])
    lane = lax.broadcasted_iota(jnp.int32, (R, 128), 1)
    acc = jnp.zeros((R, 1), jnp.float32)
    for t in range(_K + 1):
        m = jnp.min(s[0], axis=1, keepdims=True)
        if t > 0:
            acc = acc + jnp.maximum(m, 1e-12)
        if t < _K:
            hit = s[0] == m
            li = jnp.min(
                jnp.where(hit, lane, jnp.int32(999)), axis=1, keepdims=True
            )
            hf = lane == li
            for i in range(_K):
                s[i] = jnp.where(hf, s[i + 1], s[i])
            s[_K] = jnp.where(hf, INF, s[_K])
    out_ref[...] = jnp.sqrt(acc * (1.0 / _K))


def _dtm(pts):
    ptsp = jnp.zeros((_N, 8), jnp.float32).at[:, :3].set(pts)
    ptsT = ptsp.T
    out = pl.pallas_call(
        _dtm_body,
        grid=(_N // _R,),
        in_specs=[
            pl.BlockSpec((_R, 8), lambda i: (i, 0)),
            pl.BlockSpec((8, _N), lambda i: (0, 0)),
        ],
        out_specs=pl.BlockSpec((_R, 1), lambda i: (i, 0)),
        out_shape=jax.ShapeDtypeStruct((_N, 1), jnp.float32),
    )(ptsp, ptsT)
    return out[:, 0]


_NW = 32
_EPW = 1568
_EPAD = _NW * _EPW


def _edge_body(tbl_hbm, eu_hbm, ev_hbm, out_hbm, tbl_v, iu_v, iv_v, res_v):
    c = lax.axis_index("c")
    s = lax.axis_index("s")
    wid = s * 2 + c
    base = wid * _EPW
    pltpu.sync_copy(tbl_hbm, tbl_v)
    pltpu.sync_copy(eu_hbm.at[pl.ds(base, _EPW)], iu_v)
    pltpu.sync_copy(ev_hbm.at[pl.ds(base, _EPW)], iv_v)
    for i in range(_EPW // 16):
        u = iu_v[pl.ds(i * 16, 16)]
        v = iv_v[pl.ds(i * 16, 16)]
        xu = plsc.load_gather(tbl_v, [u])
        xv = plsc.load_gather(tbl_v, [v])
        yu = plsc.load_gather(tbl_v, [u + _N])
        yv = plsc.load_gather(tbl_v, [v + _N])
        zu = plsc.load_gather(tbl_v, [u + 2 * _N])
        zv = plsc.load_gather(tbl_v, [v + 2 * _N])
        fu = plsc.load_gather(tbl_v, [u + 3 * _N])
        fv = plsc.load_gather(tbl_v, [v + 3 * _N])
        dx = xu - xv
        dy = yu - yv
        dz = zu - zv
        s2 = dx * dx + dy * dy + dz * dz + 1e-12
        ib = plsc.bitcast(s2, jnp.int32)
        yb = lax.shift_right_logical(ib, jnp.int32(1)) + jnp.int32(0x1FBD1DF5)
        y = plsc.bitcast(yb, jnp.float32)
        y = 0.5 * (y + s2 / y)
        y = 0.5 * (y + s2 / y)
        y = 0.5 * (y + s2 / y)
        res_v[pl.ds(i * 16, 16)] = y + jnp.maximum(fu, fv)
    pltpu.sync_copy(res_v, out_hbm.at[pl.ds(base, _EPW)])


def _edge_call(tbl, eu, ev):
    mesh = plsc.VectorSubcoreMesh(core_axis_name="c", subcore_axis_name="s")
    run = pl.kernel(
        _edge_body,
        out_type=jax.ShapeDtypeStruct((_EPAD,), jnp.float32),
        mesh=mesh,
        compiler_params=pltpu.CompilerParams(needs_layout_passes=False),
        scratch_types=[
            pltpu.VMEM((4 * _N,), jnp.float32),
            pltpu.VMEM((_EPW,), jnp.int32),
            pltpu.VMEM((_EPW,), jnp.int32),
            pltpu.VMEM((_EPW,), jnp.float32),
        ],
    )
    return run(tbl, eu, ev)


def kernel(pts, edges):
    dtm = _dtm(pts)
    tbl = jnp.concatenate([pts, dtm[:, None]], axis=1).T.reshape(-1)
    eu = jnp.zeros((_EPAD,), jnp.int32).at[:_E].set(edges[:, 0])
    ev = jnp.zeros((_EPAD,), jnp.int32).at[:_E].set(edges[:, 1])
    out = _edge_call(tbl, eu, ev)
    return out[:_E]
